# baseline (speedup 1.0000x reference)
import functools

import jax
import jax.numpy as jnp
from jax import lax
from jax.experimental import pallas as pl
from jax.experimental.pallas import tpu as pltpu
from jax.experimental.pallas import tpu_sc as plsc

N = 10000
NFEAT = 128
NHID = 64
E = 160000
NEXT = 144

NC, NS = 2, 16
NW = NC * NS
CH = 128
EPW = 5120
NCH = EPW // CH
E_PAD = EPW * NW
NP = 10240
RPW = NP // NS
NTRASH = NP - N

ROWB = 2000

_mesh = plsc.VectorSubcoreMesh(
    core_axis_name="c", subcore_axis_name="s", num_cores=NC, num_subcores=NS)


def _tc1_body(x_ref, w_ref, o_ref):
    o_ref[...] = jnp.dot(x_ref[...], w_ref[...],
                         preferred_element_type=jnp.float32)


def _tc2_body(parts_ref, bg_ref, bgt_ref, amat_ref, ppW_ref, ppb_ref,
              pp2W_ref, pp2b_ref, ro_ref, rt_ref, pq_ref, tr_ref, cs_ref,
              ext_ref, eq_ref):
    i = pl.program_id(0)
    agg = parts_ref[0] + parts_ref[1]
    ro = jax.nn.relu(agg[:, :NHID] + bg_ref[...])
    rt = jax.nn.relu(agg[:, NHID:] + bgt_ref[...])
    ro_ref[...] = ro
    rt_ref[...] = rt
    rep = jnp.concatenate([ro, rt], axis=1)
    pq = jnp.dot(rep, amat_ref[...], preferred_element_type=jnp.float32)
    pq_ref[...] = pq
    eq = jnp.exp(pq[:, 1:2])
    eq_ref[...] = eq
    ext_ref[...] = jnp.concatenate([rt * eq, rt], axis=1)
    t1 = jnp.dot(rt, ppW_ref[...], preferred_element_type=jnp.float32)
    t1 = t1 + ppb_ref[...]
    t2 = jnp.dot(t1, pp2W_ref[...], preferred_element_type=jnp.float32)
    tr_ref[...] = jax.nn.sigmoid(t2 + pp2b_ref[...])

    @pl.when(i == 0)
    def _():
        cs_ref[...] = jnp.zeros_like(cs_ref)

    cs_ref[...] += jnp.sum(rt, axis=0, keepdims=True)


def _tc3_body(sp_ref, zs_ref, ds_ref, pq_ref, cs_ref, ro_ref, t_ref,
              o00W_ref, o00b_ref, o10W_ref, o10b_ref, o01W_ref, o01b_ref,
              o11W_ref, o11b_ref, y_ref, rep_ref):
    s = sp_ref[0] + sp_ref[1]
    ep = jnp.exp(pq_ref[...][:, :1])
    numer = ep * s[:, :NHID] - s[:, NHID:] + cs_ref[...]
    z = ep * (zs_ref[0] + zs_ref[1]) - (ds_ref[0] + ds_ref[1])
    z = z + jnp.float32(N)
    rep = numer / z + ro_ref[...]
    rep_ref[...] = rep
    y00 = jax.nn.relu(jnp.dot(rep, o00W_ref[...],
                              preferred_element_type=jnp.float32) + o00b_ref[...])
    y10 = jax.nn.relu(jnp.dot(rep, o10W_ref[...],
                              preferred_element_type=jnp.float32) + o10b_ref[...])
    y0 = jnp.dot(y00, o01W_ref[...], preferred_element_type=jnp.float32)
    y1 = jnp.dot(y10, o11W_ref[...], preferred_element_type=jnp.float32)
    y0 = y0 + o01b_ref[...]
    y1 = y1 + o11b_ref[...]
    y_ref[...] = jnp.where(t_ref[...] > 0, y1, y0)


TBITS = 25
TSIZE = 1 << TBITS


def _sc_dedup_body(slot_hbm, eid_hbm, tbl_hbm, sbuf, ebuf, sem):
    c = lax.axis_index("c")
    s = lax.axis_index("s")
    w = c * NS + s

    def chunk(k, carry):
        base = w * EPW + k * CH
        pltpu.sync_copy(slot_hbm.at[pl.ds(base, CH)], sbuf)
        pltpu.sync_copy(eid_hbm.at[pl.ds(base, CH)], ebuf)
        pltpu.async_copy(ebuf, tbl_hbm.at[sbuf], sem).wait()
        return carry
    lax.fori_loop(0, NCH, chunk, 0)


_sc_dedup = functools.partial(
    pl.kernel,
    out_type=jax.ShapeDtypeStruct((TSIZE,), jnp.int32),
    mesh=_mesh,
    scratch_types=[
        pltpu.VMEM((CH,), jnp.int32),
        pltpu.VMEM((CH,), jnp.int32),
        pltpu.SemaphoreType.DMA,
    ],
)(_sc_dedup_body)


def _sc_seg_sum_body(h_hbm, tbl_hbm, slot_hbm, key_hbm, eid_hbm, src_hbm,
                     dst_hbm, out_hbm, src2_hbm, didx, sidx, sbuf, kbuf,
                     ebuf, srcb, winb, kwb, rows, agg, sem, sem2):
    c = lax.axis_index("c")
    s = lax.axis_index("s")
    w = c * NS + s

    def zrow(r, carry):
        for g in range(NFEAT // 16):
            rows[r, pl.ds(g * 16, 16)] = jnp.zeros((16,), jnp.float32)
        return carry
    lax.fori_loop(0, CH, zrow, 0)
    for kk in range(RPW // CH):
        pltpu.sync_copy(rows, agg.at[pl.ds(s * RPW + kk * CH, CH)])
    plsc.subcore_barrier()

    def chunk(k, carry):
        base = w * EPW + k * CH
        pltpu.sync_copy(dst_hbm.at[pl.ds(base, CH)], didx)
        cp_rows = pltpu.async_copy(h_hbm.at[didx], rows, sem)
        pltpu.sync_copy(slot_hbm.at[pl.ds(base, CH)], sbuf)
        cp_win = pltpu.async_copy(tbl_hbm.at[sbuf], winb, sem2)
        pltpu.sync_copy(key_hbm.at[pl.ds(base, CH)], kbuf)
        pltpu.sync_copy(eid_hbm.at[pl.ds(base, CH)], ebuf)
        pltpu.sync_copy(src_hbm.at[pl.ds(base, CH)], srcb)
        cp_win.wait()
        pltpu.async_copy(key_hbm.at[winb], kwb, sem2).wait()
        for g in range(CH // 16):
            dsl = pl.ds(g * 16, 16)
            keep = (winb[dsl] == ebuf[dsl]) | (kwb[dsl] != kbuf[dsl])
            trash16 = N + lax.rem(ebuf[dsl], NTRASH)
            sidx[dsl] = jnp.where(keep, srcb[dsl], trash16)
        cp_rows.wait()
        pltpu.sync_copy(rows, agg.at[sidx], add=True)
        pltpu.sync_copy(sidx, src2_hbm.at[pl.ds(base, CH)])
        return carry
    lax.fori_loop(0, NCH, chunk, 0)

    plsc.subcore_barrier()
    for kk in range(RPW // CH):
        pltpu.sync_copy(agg.at[pl.ds(s * RPW + kk * CH, CH)], rows)
        pltpu.sync_copy(rows, out_hbm.at[c, pl.ds(s * RPW + kk * CH, CH)])


_seg_sum_128 = functools.partial(
    pl.kernel,
    out_type=(jax.ShapeDtypeStruct((NC, NP, NFEAT), jnp.float32),
              jax.ShapeDtypeStruct((E_PAD,), jnp.int32)),
    mesh=_mesh,
    scratch_types=[
        pltpu.VMEM((CH,), jnp.int32),
        pltpu.VMEM((CH,), jnp.int32),
        pltpu.VMEM((CH,), jnp.int32),
        pltpu.VMEM((CH,), jnp.int32),
        pltpu.VMEM((CH,), jnp.int32),
        pltpu.VMEM((CH,), jnp.int32),
        pltpu.VMEM((CH,), jnp.int32),
        pltpu.VMEM((CH,), jnp.int32),
        pltpu.VMEM((CH, NFEAT), jnp.float32),
        pltpu.VMEM_SHARED((NP, NFEAT), jnp.float32),
        pltpu.SemaphoreType.DMA,
        pltpu.SemaphoreType.DMA,
    ],
)(_sc_seg_sum_body)


def _sc_att_body(ext_hbm, eq_hbm, src_hbm, dst_hbm, out_hbm, zout_hbm,
                 dout_hbm, didx, sidx, rows, wval, ones, agg, zacc,
                 dacc, sem, sem2):
    c = lax.axis_index("c")
    s = lax.axis_index("s")
    w = c * NS + s

    def zrow(r, carry):
        for g in range(NFEAT // 16):
            rows[r, pl.ds(g * 16, 16)] = jnp.zeros((16,), jnp.float32)
        return carry
    lax.fori_loop(0, CH, zrow, 0)
    for g in range(CH // 16):
        wval[pl.ds(g * 16, 16)] = jnp.zeros((16,), jnp.float32)
        ones[pl.ds(g * 16, 16)] = jnp.ones((16,), jnp.float32)
    for kk in range(RPW // CH):
        pltpu.sync_copy(rows, agg.at[pl.ds(s * RPW + kk * CH, CH)])
        pltpu.sync_copy(wval, zacc.at[pl.ds(s * RPW + kk * CH, CH)])
        pltpu.sync_copy(wval, dacc.at[pl.ds(s * RPW + kk * CH, CH)])
    plsc.subcore_barrier()

    def chunk(k, carry):
        base = w * EPW + k * CH
        pltpu.sync_copy(dst_hbm.at[pl.ds(base, CH)], didx)
        pltpu.sync_copy(src_hbm.at[pl.ds(base, CH)], sidx)
        cp1 = pltpu.async_copy(ext_hbm.at[didx], rows, sem)
        cp2 = pltpu.async_copy(eq_hbm.at[didx], wval, sem2)
        cp1.wait()
        cp2.wait()
        pltpu.sync_copy(rows, agg.at[sidx], add=True)
        pltpu.sync_copy(wval, zacc.at[sidx], add=True)
        pltpu.sync_copy(ones, dacc.at[sidx], add=True)
        return carry
    lax.fori_loop(0, NCH, chunk, 0)

    plsc.subcore_barrier()
    for kk in range(RPW // CH):
        pltpu.sync_copy(agg.at[pl.ds(s * RPW + kk * CH, CH)], rows)
        pltpu.sync_copy(rows, out_hbm.at[c, pl.ds(s * RPW + kk * CH, CH)])
        pltpu.sync_copy(zacc.at[pl.ds(s * RPW + kk * CH, CH)], wval)
        pltpu.sync_copy(wval, zout_hbm.at[c, pl.ds(s * RPW + kk * CH, CH)])
        pltpu.sync_copy(dacc.at[pl.ds(s * RPW + kk * CH, CH)], wval)
        pltpu.sync_copy(wval, dout_hbm.at[c, pl.ds(s * RPW + kk * CH, CH)])


_sc_att = functools.partial(
    pl.kernel,
    out_type=(jax.ShapeDtypeStruct((NC, NP, NFEAT), jnp.float32),
              jax.ShapeDtypeStruct((NC, NP), jnp.float32),
              jax.ShapeDtypeStruct((NC, NP), jnp.float32)),
    mesh=_mesh,
    scratch_types=[
        pltpu.VMEM((CH,), jnp.int32),
        pltpu.VMEM((CH,), jnp.int32),
        pltpu.VMEM((CH, NFEAT), jnp.float32),
        pltpu.VMEM((CH,), jnp.float32),
        pltpu.VMEM((CH,), jnp.float32),
        pltpu.VMEM_SHARED((NP, NFEAT), jnp.float32),
        pltpu.VMEM_SHARED((NP,), jnp.float32),
        pltpu.VMEM_SHARED((NP,), jnp.float32),
        pltpu.SemaphoreType.DMA,
        pltpu.SemaphoreType.DMA,
    ],
)(_sc_att_body)


def kernel(x, adj, t, Wg, bg, Wgt, bgt, a, ppW, ppb, pp2W, pp2b,
           o00W, o00b, o10W, o10b, o01W, o01b, o11W, o11b):
    src = adj[0]
    dst = adj[1]

    pad = jnp.arange(E_PAD - E, dtype=jnp.int32)
    key = jnp.concatenate([src * N + dst, N * N + pad])
    hh = (key.astype(jnp.uint32) * jnp.uint32(2654435761)) >> jnp.uint32(4)
    slot = (hh & jnp.uint32(TSIZE - 1)).astype(jnp.int32)
    eids = jnp.arange(E_PAD, dtype=jnp.int32)
    src_p = jnp.concatenate([src, N + (pad % NTRASH)])
    dst_p = jnp.concatenate([dst, pad % N]).astype(jnp.int32)

    tbl = _sc_dedup(slot, eids)

    Wcat = jnp.concatenate([Wg, Wgt], axis=1)
    h = pl.pallas_call(
        _tc1_body,
        grid=(N // ROWB,),
        in_specs=[pl.BlockSpec((ROWB, NFEAT), lambda i: (i, 0)),
                  pl.BlockSpec((NFEAT, NFEAT), lambda i: (0, 0))],
        out_specs=pl.BlockSpec((ROWB, NFEAT), lambda i: (i, 0)),
        out_shape=jax.ShapeDtypeStruct((N, NFEAT), jnp.float32),
    )(x, Wcat)

    parts, src2 = _seg_sum_128(h, tbl, slot, key, eids, src_p, dst_p)

    amat = jnp.concatenate([a[:NFEAT], a[NFEAT:]], axis=1)
    ro, rt, pq, treatment, colsum, ext, eq = pl.pallas_call(
        _tc2_body,
        grid=(N // ROWB,),
        in_specs=[pl.BlockSpec((NC, ROWB, NFEAT), lambda i: (0, i, 0)),
                  pl.BlockSpec((1, NHID), lambda i: (0, 0)),
                  pl.BlockSpec((1, NHID), lambda i: (0, 0)),
                  pl.BlockSpec((NFEAT, 2), lambda i: (0, 0)),
                  pl.BlockSpec((NHID, NHID), lambda i: (0, 0)),
                  pl.BlockSpec((1, NHID), lambda i: (0, 0)),
                  pl.BlockSpec((NHID, 2), lambda i: (0, 0)),
                  pl.BlockSpec((1, 2), lambda i: (0, 0))],
        out_specs=[pl.BlockSpec((ROWB, NHID), lambda i: (i, 0)),
                   pl.BlockSpec((ROWB, NHID), lambda i: (i, 0)),
                   pl.BlockSpec((ROWB, 2), lambda i: (i, 0)),
                   pl.BlockSpec((ROWB, 2), lambda i: (i, 0)),
                   pl.BlockSpec((1, NHID), lambda i: (0, 0)),
                   pl.BlockSpec((ROWB, NFEAT), lambda i: (i, 0)),
                   pl.BlockSpec((ROWB, 1), lambda i: (i, 0))],
        out_shape=[jax.ShapeDtypeStruct((N, NHID), jnp.float32),
                   jax.ShapeDtypeStruct((N, NHID), jnp.float32),
                   jax.ShapeDtypeStruct((N, 2), jnp.float32),
                   jax.ShapeDtypeStruct((N, 2), jnp.float32),
                   jax.ShapeDtypeStruct((1, NHID), jnp.float32),
                   jax.ShapeDtypeStruct((N, NFEAT), jnp.float32),
                   jax.ShapeDtypeStruct((N, 1), jnp.float32)],
    )(parts, bg.reshape(1, NHID), bgt.reshape(1, NHID), amat, ppW,
      ppb.reshape(1, NHID), pp2W, pp2b.reshape(1, 2))

    sparts, zparts, dparts = _sc_att(ext, eq.reshape(-1), src2, dst_p)

    y2, rep = pl.pallas_call(
        _tc3_body,
        grid=(N // ROWB,),
        in_specs=[pl.BlockSpec((NC, ROWB, NFEAT), lambda i: (0, i, 0)),
                  pl.BlockSpec((NC, ROWB, 1), lambda i: (0, i, 0)),
                  pl.BlockSpec((NC, ROWB, 1), lambda i: (0, i, 0)),
                  pl.BlockSpec((ROWB, 2), lambda i: (i, 0)),
                  pl.BlockSpec((1, NHID), lambda i: (0, 0)),
                  pl.BlockSpec((ROWB, NHID), lambda i: (i, 0)),
                  pl.BlockSpec((ROWB, 1), lambda i: (i, 0)),
                  pl.BlockSpec((NHID, NHID), lambda i: (0, 0)),
                  pl.BlockSpec((1, NHID), lambda i: (0, 0)),
                  pl.BlockSpec((NHID, NHID), lambda i: (0, 0)),
                  pl.BlockSpec((1, NHID), lambda i: (0, 0)),
                  pl.BlockSpec((NHID, 1), lambda i: (0, 0)),
                  pl.BlockSpec((1, 1), lambda i: (0, 0)),
                  pl.BlockSpec((NHID, 1), lambda i: (0, 0)),
                  pl.BlockSpec((1, 1), lambda i: (0, 0))],
        out_specs=[pl.BlockSpec((ROWB, 1), lambda i: (i, 0)),
                   pl.BlockSpec((ROWB, NHID), lambda i: (i, 0))],
        out_shape=[jax.ShapeDtypeStruct((N, 1), jnp.float32),
                   jax.ShapeDtypeStruct((N, NHID), jnp.float32)],
    )(sparts, zparts.reshape(NC, NP, 1), dparts.reshape(NC, NP, 1), pq,
      colsum, ro, t.reshape(N, 1), o00W,
      o00b.reshape(1, NHID), o10W, o10b.reshape(1, NHID), o01W,
      o01b.reshape(1, 1), o11W, o11b.reshape(1, 1))

    return (y2.reshape(-1), rep, treatment)


# speedup vs baseline: 1.4241x; 1.4241x over previous
import jax
import jax.numpy as jnp
from jax import lax
from jax.experimental import pallas as pl
from jax.experimental.pallas import tpu as pltpu
from jax.experimental.pallas import tpu_sc as plsc
import functools

N = 10000
NFEAT = 128
NHID = 64
E = 160000

NC, NS = 2, 16
NW = NC * NS
CH = 128
EPW = 5120
NCH = EPW // CH
E_PAD = EPW * NW
EC = E_PAD // CH
NP = 10240
RPW = NP // NS
NTRASH = NP - N
TBITS = 25
TSIZE = 1 << TBITS

ROWB = 2000

_mesh = plsc.VectorSubcoreMesh(
    core_axis_name="c", subcore_axis_name="s", num_cores=NC, num_subcores=NS)


def _tc1_body(x_ref, w_ref, o_ref):
    o_ref[...] = jnp.dot(x_ref[...], w_ref[...],
                         preferred_element_type=jnp.float32)


def _tc2_body(parts_ref, bg_ref, bgt_ref, amat_ref, ppW_ref, ppb_ref,
              pp2W_ref, pp2b_ref, ro_ref, rt_ref, pq_ref, tr_ref, cs_ref,
              ext_ref, eq_ref):
    i = pl.program_id(0)
    agg = parts_ref[0] + parts_ref[1]
    ro = jax.nn.relu(agg[:, :NHID] + bg_ref[...])
    rt = jax.nn.relu(agg[:, NHID:] + bgt_ref[...])
    ro_ref[...] = ro
    rt_ref[...] = rt
    rep = jnp.concatenate([ro, rt], axis=1)
    pq = jnp.dot(rep, amat_ref[...], preferred_element_type=jnp.float32)
    pq_ref[...] = pq
    eq = jnp.exp(pq[:, 1:2])
    eq_ref[...] = eq
    ext_ref[...] = jnp.concatenate([rt * eq, rt], axis=1)
    t1 = jnp.dot(rt, ppW_ref[...], preferred_element_type=jnp.float32)
    t1 = t1 + ppb_ref[...]
    t2 = jnp.dot(t1, pp2W_ref[...], preferred_element_type=jnp.float32)
    tr_ref[...] = jax.nn.sigmoid(t2 + pp2b_ref[...])

    @pl.when(i == 0)
    def _():
        cs_ref[...] = jnp.zeros_like(cs_ref)

    cs_ref[...] += jnp.sum(rt, axis=0, keepdims=True)


def _tc3_body(sp_ref, zs_ref, ds_ref, pq_ref, cs_ref, ro_ref, t_ref,
              o00W_ref, o00b_ref, o10W_ref, o10b_ref, o01W_ref, o01b_ref,
              o11W_ref, o11b_ref, y_ref, rep_ref):
    s = sp_ref[0] + sp_ref[1]
    ep = jnp.exp(pq_ref[...][:, :1])
    numer = ep * s[:, :NHID] - s[:, NHID:] + cs_ref[...]
    z = ep * (zs_ref[0] + zs_ref[1]) - (ds_ref[0] + ds_ref[1])
    z = z + jnp.float32(N)
    rep = numer / z + ro_ref[...]
    rep_ref[...] = rep
    y00 = jax.nn.relu(jnp.dot(rep, o00W_ref[...],
                              preferred_element_type=jnp.float32) + o00b_ref[...])
    y10 = jax.nn.relu(jnp.dot(rep, o10W_ref[...],
                              preferred_element_type=jnp.float32) + o10b_ref[...])
    y0 = jnp.dot(y00, o01W_ref[...], preferred_element_type=jnp.float32)
    y1 = jnp.dot(y10, o11W_ref[...], preferred_element_type=jnp.float32)
    y0 = y0 + o01b_ref[...]
    y1 = y1 + o11b_ref[...]
    y_ref[...] = jnp.where(t_ref[...] > 0, y1, y0)


_IOTA16 = lambda: lax.iota(jnp.int32, 16)


def _sc_dedup_body(slot_hbm, tbl_hbm, slotall, slotc, ebuf, s0, s1, s2, s3):
    c = lax.axis_index("c")
    s = lax.axis_index("s")
    w = c * NS + s
    sems = (s0, s1, s2, s3)

    pltpu.sync_copy(slot_hbm.at[pl.ds(w * EPW, EPW)], slotall)
    iota16 = _IOTA16()
    base_w = w * EPW
    descs = [None] * 4
    for k in range(NCH):
        r = k % 4
        if descs[r] is not None:
            descs[r].wait()
        for g in range(CH // 16):
            dsl = pl.ds(g * 16, 16)
            slotc[r, dsl] = slotall[pl.ds(k * CH + g * 16, 16)]
            ebuf[r, dsl] = base_w + (k * CH + g * 16) + iota16
        descs[r] = pltpu.async_copy(
            ebuf.at[r], tbl_hbm.at[slotc.at[r]], sems[r])
    for r in range(4):
        descs[r].wait()


_sc_dedup = functools.partial(
    pl.kernel,
    out_type=jax.ShapeDtypeStruct((TSIZE,), jnp.int32),
    mesh=_mesh,
    scratch_types=[
        pltpu.VMEM((EPW,), jnp.int32),
        pltpu.VMEM((4, CH), jnp.int32),
        pltpu.VMEM((4, CH), jnp.int32),
        pltpu.SemaphoreType.DMA,
        pltpu.SemaphoreType.DMA,
        pltpu.SemaphoreType.DMA,
        pltpu.SemaphoreType.DMA,
    ],
)(_sc_dedup_body)


def _zero_rows3(rows, r, ncols):
    def body(i, carry):
        for g in range(ncols // 16):
            rows[r, i, pl.ds(g * 16, 16)] = jnp.zeros((16,), jnp.float32)
        return carry
    lax.fori_loop(0, CH, body, 0)


def _sc_resolve_body(tbl_hbm, slot_hbm, key_hbm, src_hbm, src2_hbm,
                     slotall, keyall, srcall, winall, kwall, sidxall,
                     semw, semk):
    c = lax.axis_index("c")
    s = lax.axis_index("s")
    w = c * NS + s
    wflat = pl.ds(w * EPW, EPW)

    pltpu.sync_copy(slot_hbm.at[wflat], slotall)
    pltpu.sync_copy(key_hbm.at[wflat], keyall)
    pltpu.sync_copy(src_hbm.at[wflat], srcall)

    def fire_win(k, carry):
        ck = pl.ds(k * CH, CH)
        pltpu.async_copy(tbl_hbm.at[slotall.at[ck]], winall.at[ck], semw)
        return carry
    lax.fori_loop(0, NCH, fire_win, 0)
    pltpu.make_async_copy(slot_hbm.at[wflat], winall, semw).wait()

    def fire_kw(k, carry):
        ck = pl.ds(k * CH, CH)
        pltpu.async_copy(key_hbm.at[winall.at[ck]], kwall.at[ck], semk)
        return carry
    lax.fori_loop(0, NCH, fire_kw, 0)
    pltpu.make_async_copy(key_hbm.at[wflat], kwall, semk).wait()

    iota16 = _IOTA16()
    base_w = w * EPW

    def resolve(k, carry):
        for g in range(CH // 16):
            dsl = pl.ds(g * 16, 16)
            fsl = pl.ds(k * CH + g * 16, 16)
            eid16 = base_w + (k * CH + g * 16) + iota16
            keep = (winall[fsl] == eid16) | (kwall[fsl] != keyall[fsl])
            trash16 = N + lax.rem(eid16, NTRASH)
            sidxall[k, dsl] = jnp.where(keep, srcall[fsl], trash16)
        return carry
    lax.fori_loop(0, NCH, resolve, 0)
    pltpu.sync_copy(sidxall, src2_hbm.at[pl.ds(w * NCH, NCH)])


_sc_resolve = functools.partial(
    pl.kernel,
    out_type=jax.ShapeDtypeStruct((EC, CH), jnp.int32),
    mesh=_mesh,
    scratch_types=[
        pltpu.VMEM((EPW,), jnp.int32),
        pltpu.VMEM((EPW,), jnp.int32),
        pltpu.VMEM((EPW,), jnp.int32),
        pltpu.VMEM((EPW,), jnp.int32),
        pltpu.VMEM((EPW,), jnp.int32),
        pltpu.VMEM((NCH, CH), jnp.int32),
        pltpu.SemaphoreType.DMA,
        pltpu.SemaphoreType.DMA,
    ],
)(_sc_resolve_body)


def _sc_seg_sum_body(h_hbm, src2_hbm, dst_hbm, out_hbm, dstall, srcall,
                     rows, agg, sr0, sr1):
    c = lax.axis_index("c")
    s = lax.axis_index("s")
    w = c * NS + s

    pltpu.sync_copy(dst_hbm.at[pl.ds(w * EPW, EPW)], dstall)
    pltpu.sync_copy(src2_hbm.at[pl.ds(w * NCH, NCH)], srcall)

    _zero_rows3(rows, 0, NFEAT)
    for kk in range(RPW // CH):
        pltpu.sync_copy(rows.at[0], agg.at[pl.ds(s * RPW + kk * CH, CH)])
    plsc.subcore_barrier()

    sems = (sr0, sr1)
    descs = [None, None]
    descs[0] = pltpu.async_copy(
        h_hbm.at[dstall.at[pl.ds(0, CH)]], rows.at[0], sems[0])
    for k in range(NCH):
        r = k & 1
        if k + 1 < NCH:
            rn = (k + 1) & 1
            descs[rn] = pltpu.async_copy(
                h_hbm.at[dstall.at[pl.ds((k + 1) * CH, CH)]],
                rows.at[rn], sems[rn])
        descs[r].wait()
        pltpu.sync_copy(rows.at[r], agg.at[srcall.at[k]], add=True)

    plsc.subcore_barrier()
    for kk in range(RPW // CH):
        pltpu.sync_copy(agg.at[pl.ds(s * RPW + kk * CH, CH)], rows.at[0])
        pltpu.sync_copy(rows.at[0], out_hbm.at[c, pl.ds(s * RPW + kk * CH, CH)])


_sc_seg_sum = functools.partial(
    pl.kernel,
    out_type=jax.ShapeDtypeStruct((NC, NP, NFEAT), jnp.float32),
    mesh=_mesh,
    scratch_types=[
        pltpu.VMEM((EPW,), jnp.int32),
        pltpu.VMEM((NCH, CH), jnp.int32),
        pltpu.VMEM((2, CH, NFEAT), jnp.float32),
        pltpu.VMEM_SHARED((NP, NFEAT), jnp.float32),
        pltpu.SemaphoreType.DMA,
        pltpu.SemaphoreType.DMA,
    ],
)(_sc_seg_sum_body)


def _sc_att_body(ext_hbm, eq_hbm, src_hbm, dst_hbm, out_hbm, zout_hbm,
                 dout_hbm, dstall, srcall, eqall, ones, rows, agg, zacc,
                 dacc, sq0, sq1, sr0, sr1):
    c = lax.axis_index("c")
    s = lax.axis_index("s")
    w = c * NS + s

    pltpu.sync_copy(dst_hbm.at[pl.ds(w * EPW, EPW)], dstall)
    pltpu.sync_copy(src_hbm.at[pl.ds(w * NCH, NCH)], srcall)

    _zero_rows3(rows, 0, NFEAT)
    for g in range(CH // 16):
        ones[pl.ds(g * 16, 16)] = jnp.ones((16,), jnp.float32)
    for kk in range(RPW // CH):
        pltpu.sync_copy(rows.at[0], agg.at[pl.ds(s * RPW + kk * CH, CH)])
        pltpu.sync_copy(rows.at[0, 0], zacc.at[pl.ds(s * RPW + kk * CH, CH)])
        pltpu.sync_copy(rows.at[0, 0], dacc.at[pl.ds(s * RPW + kk * CH, CH)])

    plsc.subcore_barrier()

    sems = (sr0, sr1)
    qsems = (sq0, sq1)
    descs = [None, None]
    qdescs = [None, None]
    descs[0] = pltpu.async_copy(
        ext_hbm.at[dstall.at[pl.ds(0, CH)]], rows.at[0], sems[0])
    qdescs[0] = pltpu.async_copy(
        eq_hbm.at[dstall.at[pl.ds(0, CH)]], eqall.at[0], qsems[0])
    for k in range(NCH):
        r = k & 1
        if k + 1 < NCH:
            rn = (k + 1) & 1
            nsl = dstall.at[pl.ds((k + 1) * CH, CH)]
            descs[rn] = pltpu.async_copy(
                ext_hbm.at[nsl], rows.at[rn], sems[rn])
            qdescs[rn] = pltpu.async_copy(
                eq_hbm.at[nsl], eqall.at[rn], qsems[rn])
        descs[r].wait()
        qdescs[r].wait()
        pltpu.sync_copy(rows.at[r], agg.at[srcall.at[k]], add=True)
        pltpu.sync_copy(eqall.at[r], zacc.at[srcall.at[k]], add=True)
        pltpu.sync_copy(ones, dacc.at[srcall.at[k]], add=True)

    plsc.subcore_barrier()
    for kk in range(RPW // CH):
        rowsl = pl.ds(s * RPW + kk * CH, CH)
        pltpu.sync_copy(agg.at[rowsl], rows.at[0])
        pltpu.sync_copy(rows.at[0], out_hbm.at[c, rowsl])
        pltpu.sync_copy(zacc.at[rowsl], eqall.at[0])
        pltpu.sync_copy(eqall.at[0], zout_hbm.at[c, rowsl])
        pltpu.sync_copy(dacc.at[rowsl], eqall.at[0])
        pltpu.sync_copy(eqall.at[0], dout_hbm.at[c, rowsl])


_sc_att = functools.partial(
    pl.kernel,
    out_type=(jax.ShapeDtypeStruct((NC, NP, NFEAT), jnp.float32),
              jax.ShapeDtypeStruct((NC, NP), jnp.float32),
              jax.ShapeDtypeStruct((NC, NP), jnp.float32)),
    mesh=_mesh,
    scratch_types=[
        pltpu.VMEM((EPW,), jnp.int32),
        pltpu.VMEM((NCH, CH), jnp.int32),
        pltpu.VMEM((2, CH), jnp.float32),
        pltpu.VMEM((CH,), jnp.float32),
        pltpu.VMEM((2, CH, NFEAT), jnp.float32),
        pltpu.VMEM_SHARED((NP, NFEAT), jnp.float32),
        pltpu.VMEM_SHARED((NP,), jnp.float32),
        pltpu.VMEM_SHARED((NP,), jnp.float32),
        pltpu.SemaphoreType.DMA,
        pltpu.SemaphoreType.DMA,
        pltpu.SemaphoreType.DMA,
        pltpu.SemaphoreType.DMA,
    ],
)(_sc_att_body)


def kernel(x, adj, t, Wg, bg, Wgt, bgt, a, ppW, ppb, pp2W, pp2b,
           o00W, o00b, o10W, o10b, o01W, o01b, o11W, o11b):
    src = adj[0]
    dst = adj[1]

    pad = jnp.arange(E_PAD - E, dtype=jnp.int32)
    key = jnp.concatenate([src * N + dst, N * N + pad])
    hh = (key.astype(jnp.uint32) * jnp.uint32(2654435761)) >> jnp.uint32(4)
    slot = (hh & jnp.uint32(TSIZE - 1)).astype(jnp.int32)
    src_p = jnp.concatenate([src, N + (pad % NTRASH)])
    dst_p = jnp.concatenate([dst, pad % N]).astype(jnp.int32)

    tbl = _sc_dedup(slot)
    src2res = _sc_resolve(tbl, slot, key, src_p)

    Wcat = jnp.concatenate([Wg, Wgt], axis=1)
    h = pl.pallas_call(
        _tc1_body,
        grid=(N // ROWB,),
        in_specs=[pl.BlockSpec((ROWB, NFEAT), lambda i: (i, 0)),
                  pl.BlockSpec((NFEAT, NFEAT), lambda i: (0, 0))],
        out_specs=pl.BlockSpec((ROWB, NFEAT), lambda i: (i, 0)),
        out_shape=jax.ShapeDtypeStruct((N, NFEAT), jnp.float32),
    )(x, Wcat)

    parts = _sc_seg_sum(h, src2res, dst_p)

    amat = jnp.concatenate([a[:NFEAT], a[NFEAT:]], axis=1)
    ro, rt, pq, treatment, colsum, ext, eq = pl.pallas_call(
        _tc2_body,
        grid=(N // ROWB,),
        in_specs=[pl.BlockSpec((NC, ROWB, NFEAT), lambda i: (0, i, 0)),
                  pl.BlockSpec((1, NHID), lambda i: (0, 0)),
                  pl.BlockSpec((1, NHID), lambda i: (0, 0)),
                  pl.BlockSpec((NFEAT, 2), lambda i: (0, 0)),
                  pl.BlockSpec((NHID, NHID), lambda i: (0, 0)),
                  pl.BlockSpec((1, NHID), lambda i: (0, 0)),
                  pl.BlockSpec((NHID, 2), lambda i: (0, 0)),
                  pl.BlockSpec((1, 2), lambda i: (0, 0))],
        out_specs=[pl.BlockSpec((ROWB, NHID), lambda i: (i, 0)),
                   pl.BlockSpec((ROWB, NHID), lambda i: (i, 0)),
                   pl.BlockSpec((ROWB, 2), lambda i: (i, 0)),
                   pl.BlockSpec((ROWB, 2), lambda i: (i, 0)),
                   pl.BlockSpec((1, NHID), lambda i: (0, 0)),
                   pl.BlockSpec((ROWB, NFEAT), lambda i: (i, 0)),
                   pl.BlockSpec((ROWB, 1), lambda i: (i, 0))],
        out_shape=[jax.ShapeDtypeStruct((N, NHID), jnp.float32),
                   jax.ShapeDtypeStruct((N, NHID), jnp.float32),
                   jax.ShapeDtypeStruct((N, 2), jnp.float32),
                   jax.ShapeDtypeStruct((N, 2), jnp.float32),
                   jax.ShapeDtypeStruct((1, NHID), jnp.float32),
                   jax.ShapeDtypeStruct((N, NFEAT), jnp.float32),
                   jax.ShapeDtypeStruct((N, 1), jnp.float32)],
    )(parts, bg.reshape(1, NHID), bgt.reshape(1, NHID), amat, ppW,
      ppb.reshape(1, NHID), pp2W, pp2b.reshape(1, 2))

    sparts, zparts, dparts = _sc_att(ext, eq.reshape(-1), src2res, dst_p)

    y2, rep = pl.pallas_call(
        _tc3_body,
        grid=(N // ROWB,),
        in_specs=[pl.BlockSpec((NC, ROWB, NFEAT), lambda i: (0, i, 0)),
                  pl.BlockSpec((NC, ROWB, 1), lambda i: (0, i, 0)),
                  pl.BlockSpec((NC, ROWB, 1), lambda i: (0, i, 0)),
                  pl.BlockSpec((ROWB, 2), lambda i: (i, 0)),
                  pl.BlockSpec((1, NHID), lambda i: (0, 0)),
                  pl.BlockSpec((ROWB, NHID), lambda i: (i, 0)),
                  pl.BlockSpec((ROWB, 1), lambda i: (i, 0)),
                  pl.BlockSpec((NHID, NHID), lambda i: (0, 0)),
                  pl.BlockSpec((1, NHID), lambda i: (0, 0)),
                  pl.BlockSpec((NHID, NHID), lambda i: (0, 0)),
                  pl.BlockSpec((1, NHID), lambda i: (0, 0)),
                  pl.BlockSpec((NHID, 1), lambda i: (0, 0)),
                  pl.BlockSpec((1, 1), lambda i: (0, 0)),
                  pl.BlockSpec((NHID, 1), lambda i: (0, 0)),
                  pl.BlockSpec((1, 1), lambda i: (0, 0))],
        out_specs=[pl.BlockSpec((ROWB, 1), lambda i: (i, 0)),
                   pl.BlockSpec((ROWB, NHID), lambda i: (i, 0))],
        out_shape=[jax.ShapeDtypeStruct((N, 1), jnp.float32),
                   jax.ShapeDtypeStruct((N, NHID), jnp.float32)],
    )(sparts, zparts.reshape(NC, NP, 1), dparts.reshape(NC, NP, 1), pq,
      colsum, ro, t.reshape(N, 1), o00W,
      o00b.reshape(1, NHID), o10W, o10b.reshape(1, NHID), o01W,
      o01b.reshape(1, 1), o11W, o11b.reshape(1, 1))

    return (y2.reshape(-1), rep, treatment)


# speedup vs baseline: 1.4307x; 1.0046x over previous
import jax
import jax.numpy as jnp
from jax import lax
from jax.experimental import pallas as pl
from jax.experimental.pallas import tpu as pltpu
from jax.experimental.pallas import tpu_sc as plsc
import functools

N = 10000
NFEAT = 128
NHID = 64
E = 160000

NC, NS = 2, 16
NW = NC * NS
CH = 128
EPW = 5120
NCH = EPW // CH
E_PAD = EPW * NW
EC = E_PAD // CH
NP = 10240
RPW = NP // NS
NTRASH = NP - N
TBITS = 25
TSIZE = 1 << TBITS

ROWB = 2000

_mesh = plsc.VectorSubcoreMesh(
    core_axis_name="c", subcore_axis_name="s", num_cores=NC, num_subcores=NS)


def _tc1_body(x_ref, w_ref, o_ref):
    o_ref[...] = jnp.dot(x_ref[...], w_ref[...],
                         preferred_element_type=jnp.float32)


def _tc2_body(parts_ref, bg_ref, bgt_ref, amat_ref, ppW_ref, ppb_ref,
              pp2W_ref, pp2b_ref, ro_ref, rt_ref, pq_ref, tr_ref, cs_ref,
              ext_ref, eq_ref):
    i = pl.program_id(0)
    agg = parts_ref[0] + parts_ref[1]
    ro = jax.nn.relu(agg[:, :NHID] + bg_ref[...])
    rt = jax.nn.relu(agg[:, NHID:] + bgt_ref[...])
    ro_ref[...] = ro
    rt_ref[...] = rt
    rep = jnp.concatenate([ro, rt], axis=1)
    pq = jnp.dot(rep, amat_ref[...], preferred_element_type=jnp.float32)
    pq_ref[...] = pq
    eq = jnp.exp(pq[:, 1:2])
    eq_ref[...] = eq
    ext_ref[...] = jnp.concatenate([rt * eq, rt], axis=1)
    t1 = jnp.dot(rt, ppW_ref[...], preferred_element_type=jnp.float32)
    t1 = t1 + ppb_ref[...]
    t2 = jnp.dot(t1, pp2W_ref[...], preferred_element_type=jnp.float32)
    tr_ref[...] = jax.nn.sigmoid(t2 + pp2b_ref[...])

    @pl.when(i == 0)
    def _():
        cs_ref[...] = jnp.zeros_like(cs_ref)

    cs_ref[...] += jnp.sum(rt, axis=0, keepdims=True)


def _tc3_body(sp_ref, zs_ref, ds_ref, pq_ref, cs_ref, ro_ref, t_ref,
              o00W_ref, o00b_ref, o10W_ref, o10b_ref, o01W_ref, o01b_ref,
              o11W_ref, o11b_ref, y_ref, rep_ref):
    s = sp_ref[0] + sp_ref[1]
    ep = jnp.exp(pq_ref[...][:, :1])
    numer = ep * s[:, :NHID] - s[:, NHID:] + cs_ref[...]
    z = ep * (zs_ref[0] + zs_ref[1]) - (ds_ref[0] + ds_ref[1])
    z = z + jnp.float32(N)
    rep = numer / z + ro_ref[...]
    rep_ref[...] = rep
    y00 = jax.nn.relu(jnp.dot(rep, o00W_ref[...],
                              preferred_element_type=jnp.float32) + o00b_ref[...])
    y10 = jax.nn.relu(jnp.dot(rep, o10W_ref[...],
                              preferred_element_type=jnp.float32) + o10b_ref[...])
    y0 = jnp.dot(y00, o01W_ref[...], preferred_element_type=jnp.float32)
    y1 = jnp.dot(y10, o11W_ref[...], preferred_element_type=jnp.float32)
    y0 = y0 + o01b_ref[...]
    y1 = y1 + o11b_ref[...]
    y_ref[...] = jnp.where(t_ref[...] > 0, y1, y0)


_IOTA16 = lambda: lax.iota(jnp.int32, 16)


def _sc_dedup_body(slot_hbm, tbl_hbm, slotall, slotc, ebuf,
                   s0, s1, s2, s3, s4, s5, s6, s7):
    c = lax.axis_index("c")
    s = lax.axis_index("s")
    w = c * NS + s
    sems = (s0, s1, s2, s3, s4, s5, s6, s7)

    pltpu.sync_copy(slot_hbm.at[pl.ds(w * EPW, EPW)], slotall)
    iota16 = _IOTA16()
    base_w = w * EPW
    descs = [None] * 8
    for k in range(NCH):
        r = k % 8
        if descs[r] is not None:
            descs[r].wait()
        for g in range(CH // 16):
            dsl = pl.ds(g * 16, 16)
            slotc[r, dsl] = slotall[pl.ds(k * CH + g * 16, 16)]
            ebuf[r, dsl] = base_w + (k * CH + g * 16) + iota16
        descs[r] = pltpu.async_copy(
            ebuf.at[r], tbl_hbm.at[slotc.at[r]], sems[r])
    for r in range(8):
        if descs[r] is not None:
            descs[r].wait()


_sc_dedup = functools.partial(
    pl.kernel,
    out_type=jax.ShapeDtypeStruct((TSIZE,), jnp.int32),
    mesh=_mesh,
    scratch_types=[
        pltpu.VMEM((EPW,), jnp.int32),
        pltpu.VMEM((8, CH), jnp.int32),
        pltpu.VMEM((8, CH), jnp.int32),
        pltpu.SemaphoreType.DMA,
        pltpu.SemaphoreType.DMA,
        pltpu.SemaphoreType.DMA,
        pltpu.SemaphoreType.DMA,
        pltpu.SemaphoreType.DMA,
        pltpu.SemaphoreType.DMA,
        pltpu.SemaphoreType.DMA,
        pltpu.SemaphoreType.DMA,
    ],
)(_sc_dedup_body)


def _zero_rows3(rows, r, ncols):
    def body(i, carry):
        for g in range(ncols // 16):
            rows[r, i, pl.ds(g * 16, 16)] = jnp.zeros((16,), jnp.float32)
        return carry
    lax.fori_loop(0, CH, body, 0)


def _sc_resolve_body(tbl_hbm, slot_hbm, key_hbm, src_hbm, src2_hbm,
                     slotall, keyall, srcall, winall, kwall, sidxall,
                     semw, semk):
    c = lax.axis_index("c")
    s = lax.axis_index("s")
    w = c * NS + s
    wflat = pl.ds(w * EPW, EPW)

    pltpu.sync_copy(slot_hbm.at[wflat], slotall)
    pltpu.sync_copy(key_hbm.at[wflat], keyall)
    pltpu.sync_copy(src_hbm.at[wflat], srcall)

    def fire_win(k, carry):
        ck = pl.ds(k * CH, CH)
        pltpu.async_copy(tbl_hbm.at[slotall.at[ck]], winall.at[ck], semw)
        return carry
    lax.fori_loop(0, NCH, fire_win, 0)
    pltpu.make_async_copy(slot_hbm.at[wflat], winall, semw).wait()

    def fire_kw(k, carry):
        ck = pl.ds(k * CH, CH)
        pltpu.async_copy(key_hbm.at[winall.at[ck]], kwall.at[ck], semk)
        return carry
    lax.fori_loop(0, NCH, fire_kw, 0)
    pltpu.make_async_copy(key_hbm.at[wflat], kwall, semk).wait()

    iota16 = _IOTA16()
    base_w = w * EPW

    def resolve(k, carry):
        for g in range(CH // 16):
            dsl = pl.ds(g * 16, 16)
            fsl = pl.ds(k * CH + g * 16, 16)
            eid16 = base_w + (k * CH + g * 16) + iota16
            keep = (winall[fsl] == eid16) | (kwall[fsl] != keyall[fsl])
            trash16 = N + lax.rem(eid16, NTRASH)
            sidxall[k, dsl] = jnp.where(keep, srcall[fsl], trash16)
        return carry
    lax.fori_loop(0, NCH, resolve, 0)
    pltpu.sync_copy(sidxall, src2_hbm.at[pl.ds(w * NCH, NCH)])


_sc_resolve = functools.partial(
    pl.kernel,
    out_type=jax.ShapeDtypeStruct((EC, CH), jnp.int32),
    mesh=_mesh,
    scratch_types=[
        pltpu.VMEM((EPW,), jnp.int32),
        pltpu.VMEM((EPW,), jnp.int32),
        pltpu.VMEM((EPW,), jnp.int32),
        pltpu.VMEM((EPW,), jnp.int32),
        pltpu.VMEM((EPW,), jnp.int32),
        pltpu.VMEM((NCH, CH), jnp.int32),
        pltpu.SemaphoreType.DMA,
        pltpu.SemaphoreType.DMA,
    ],
)(_sc_resolve_body)


def _sc_seg_sum_body(h_hbm, src2_hbm, dst_hbm, out_hbm, dstall, srcall,
                     rows, agg, sr0, sr1):
    c = lax.axis_index("c")
    s = lax.axis_index("s")
    w = c * NS + s

    pltpu.sync_copy(dst_hbm.at[pl.ds(w * EPW, EPW)], dstall)
    pltpu.sync_copy(src2_hbm.at[pl.ds(w * NCH, NCH)], srcall)

    _zero_rows3(rows, 0, NFEAT)
    for kk in range(RPW // CH):
        pltpu.sync_copy(rows.at[0], agg.at[pl.ds(s * RPW + kk * CH, CH)])
    plsc.subcore_barrier()

    sems = (sr0, sr1)
    descs = [None, None]
    descs[0] = pltpu.async_copy(
        h_hbm.at[dstall.at[pl.ds(0, CH)]], rows.at[0], sems[0])
    for k in range(NCH):
        r = k & 1
        if k + 1 < NCH:
            rn = (k + 1) & 1
            descs[rn] = pltpu.async_copy(
                h_hbm.at[dstall.at[pl.ds((k + 1) * CH, CH)]],
                rows.at[rn], sems[rn])
        descs[r].wait()
        pltpu.sync_copy(rows.at[r], agg.at[srcall.at[k]], add=True)

    plsc.subcore_barrier()
    for kk in range(RPW // CH):
        pltpu.sync_copy(agg.at[pl.ds(s * RPW + kk * CH, CH)], rows.at[0])
        pltpu.sync_copy(rows.at[0], out_hbm.at[c, pl.ds(s * RPW + kk * CH, CH)])


_sc_seg_sum = functools.partial(
    pl.kernel,
    out_type=jax.ShapeDtypeStruct((NC, NP, NFEAT), jnp.float32),
    mesh=_mesh,
    scratch_types=[
        pltpu.VMEM((EPW,), jnp.int32),
        pltpu.VMEM((NCH, CH), jnp.int32),
        pltpu.VMEM((2, CH, NFEAT), jnp.float32),
        pltpu.VMEM_SHARED((NP, NFEAT), jnp.float32),
        pltpu.SemaphoreType.DMA,
        pltpu.SemaphoreType.DMA,
    ],
)(_sc_seg_sum_body)


def _sc_att_body(ext_hbm, eq_hbm, src_hbm, dst_hbm, out_hbm, zout_hbm,
                 dout_hbm, dstall, srcall, eqall, ones, rows, agg, zacc,
                 dacc, sq0, sq1, sr0, sr1):
    c = lax.axis_index("c")
    s = lax.axis_index("s")
    w = c * NS + s

    pltpu.sync_copy(dst_hbm.at[pl.ds(w * EPW, EPW)], dstall)
    pltpu.sync_copy(src_hbm.at[pl.ds(w * NCH, NCH)], srcall)

    _zero_rows3(rows, 0, NFEAT)
    for g in range(CH // 16):
        ones[pl.ds(g * 16, 16)] = jnp.ones((16,), jnp.float32)
    for kk in range(RPW // CH):
        pltpu.sync_copy(rows.at[0], agg.at[pl.ds(s * RPW + kk * CH, CH)])
        pltpu.sync_copy(rows.at[0, 0], zacc.at[pl.ds(s * RPW + kk * CH, CH)])
        pltpu.sync_copy(rows.at[0, 0], dacc.at[pl.ds(s * RPW + kk * CH, CH)])

    plsc.subcore_barrier()

    sems = (sr0, sr1)
    qsems = (sq0, sq1)
    descs = [None, None]
    qdescs = [None, None]
    descs[0] = pltpu.async_copy(
        ext_hbm.at[dstall.at[pl.ds(0, CH)]], rows.at[0], sems[0])
    qdescs[0] = pltpu.async_copy(
        eq_hbm.at[dstall.at[pl.ds(0, CH)]], eqall.at[0], qsems[0])
    for k in range(NCH):
        r = k & 1
        if k + 1 < NCH:
            rn = (k + 1) & 1
            nsl = dstall.at[pl.ds((k + 1) * CH, CH)]
            descs[rn] = pltpu.async_copy(
                ext_hbm.at[nsl], rows.at[rn], sems[rn])
            qdescs[rn] = pltpu.async_copy(
                eq_hbm.at[nsl], eqall.at[rn], qsems[rn])
        qdescs[r].wait()
        pltpu.sync_copy(eqall.at[r], zacc.at[srcall.at[k]], add=True)
        pltpu.sync_copy(ones, dacc.at[srcall.at[k]], add=True)
        descs[r].wait()
        pltpu.sync_copy(rows.at[r], agg.at[srcall.at[k]], add=True)

    plsc.subcore_barrier()
    for kk in range(RPW // CH):
        rowsl = pl.ds(s * RPW + kk * CH, CH)
        pltpu.sync_copy(agg.at[rowsl], rows.at[0])
        pltpu.sync_copy(rows.at[0], out_hbm.at[c, rowsl])
        pltpu.sync_copy(zacc.at[rowsl], eqall.at[0])
        pltpu.sync_copy(eqall.at[0], zout_hbm.at[c, rowsl])
        pltpu.sync_copy(dacc.at[rowsl], eqall.at[0])
        pltpu.sync_copy(eqall.at[0], dout_hbm.at[c, rowsl])


_sc_att = functools.partial(
    pl.kernel,
    out_type=(jax.ShapeDtypeStruct((NC, NP, NFEAT), jnp.float32),
              jax.ShapeDtypeStruct((NC, NP), jnp.float32),
              jax.ShapeDtypeStruct((NC, NP), jnp.float32)),
    mesh=_mesh,
    scratch_types=[
        pltpu.VMEM((EPW,), jnp.int32),
        pltpu.VMEM((NCH, CH), jnp.int32),
        pltpu.VMEM((2, CH), jnp.float32),
        pltpu.VMEM((CH,), jnp.float32),
        pltpu.VMEM((2, CH, NFEAT), jnp.float32),
        pltpu.VMEM_SHARED((NP, NFEAT), jnp.float32),
        pltpu.VMEM_SHARED((NP,), jnp.float32),
        pltpu.VMEM_SHARED((NP,), jnp.float32),
        pltpu.SemaphoreType.DMA,
        pltpu.SemaphoreType.DMA,
        pltpu.SemaphoreType.DMA,
        pltpu.SemaphoreType.DMA,
    ],
)(_sc_att_body)


def kernel(x, adj, t, Wg, bg, Wgt, bgt, a, ppW, ppb, pp2W, pp2b,
           o00W, o00b, o10W, o10b, o01W, o01b, o11W, o11b):
    src = adj[0]
    dst = adj[1]

    pad = jnp.arange(E_PAD - E, dtype=jnp.int32)
    key = jnp.concatenate([src * N + dst, N * N + pad])
    hh = (key.astype(jnp.uint32) * jnp.uint32(2654435761)) >> jnp.uint32(4)
    slot = (hh & jnp.uint32(TSIZE - 1)).astype(jnp.int32)
    src_p = jnp.concatenate([src, N + (pad % NTRASH)])
    dst_p = jnp.concatenate([dst, pad % N]).astype(jnp.int32)

    tbl = _sc_dedup(slot)
    src2res = _sc_resolve(tbl, slot, key, src_p)

    Wcat = jnp.concatenate([Wg, Wgt], axis=1)
    h = pl.pallas_call(
        _tc1_body,
        grid=(N // ROWB,),
        in_specs=[pl.BlockSpec((ROWB, NFEAT), lambda i: (i, 0)),
                  pl.BlockSpec((NFEAT, NFEAT), lambda i: (0, 0))],
        out_specs=pl.BlockSpec((ROWB, NFEAT), lambda i: (i, 0)),
        out_shape=jax.ShapeDtypeStruct((N, NFEAT), jnp.float32),
    )(x, Wcat)

    parts = _sc_seg_sum(h, src2res, dst_p)

    amat = jnp.concatenate([a[:NFEAT], a[NFEAT:]], axis=1)
    ro, rt, pq, treatment, colsum, ext, eq = pl.pallas_call(
        _tc2_body,
        grid=(N // ROWB,),
        in_specs=[pl.BlockSpec((NC, ROWB, NFEAT), lambda i: (0, i, 0)),
                  pl.BlockSpec((1, NHID), lambda i: (0, 0)),
                  pl.BlockSpec((1, NHID), lambda i: (0, 0)),
                  pl.BlockSpec((NFEAT, 2), lambda i: (0, 0)),
                  pl.BlockSpec((NHID, NHID), lambda i: (0, 0)),
                  pl.BlockSpec((1, NHID), lambda i: (0, 0)),
                  pl.BlockSpec((NHID, 2), lambda i: (0, 0)),
                  pl.BlockSpec((1, 2), lambda i: (0, 0))],
        out_specs=[pl.BlockSpec((ROWB, NHID), lambda i: (i, 0)),
                   pl.BlockSpec((ROWB, NHID), lambda i: (i, 0)),
                   pl.BlockSpec((ROWB, 2), lambda i: (i, 0)),
                   pl.BlockSpec((ROWB, 2), lambda i: (i, 0)),
                   pl.BlockSpec((1, NHID), lambda i: (0, 0)),
                   pl.BlockSpec((ROWB, NFEAT), lambda i: (i, 0)),
                   pl.BlockSpec((ROWB, 1), lambda i: (i, 0))],
        out_shape=[jax.ShapeDtypeStruct((N, NHID), jnp.float32),
                   jax.ShapeDtypeStruct((N, NHID), jnp.float32),
                   jax.ShapeDtypeStruct((N, 2), jnp.float32),
                   jax.ShapeDtypeStruct((N, 2), jnp.float32),
                   jax.ShapeDtypeStruct((1, NHID), jnp.float32),
                   jax.ShapeDtypeStruct((N, NFEAT), jnp.float32),
                   jax.ShapeDtypeStruct((N, 1), jnp.float32)],
    )(parts, bg.reshape(1, NHID), bgt.reshape(1, NHID), amat, ppW,
      ppb.reshape(1, NHID), pp2W, pp2b.reshape(1, 2))

    sparts, zparts, dparts = _sc_att(ext, eq.reshape(-1), src2res, dst_p)

    y2, rep = pl.pallas_call(
        _tc3_body,
        grid=(N // ROWB,),
        in_specs=[pl.BlockSpec((NC, ROWB, NFEAT), lambda i: (0, i, 0)),
                  pl.BlockSpec((NC, ROWB, 1), lambda i: (0, i, 0)),
                  pl.BlockSpec((NC, ROWB, 1), lambda i: (0, i, 0)),
                  pl.BlockSpec((ROWB, 2), lambda i: (i, 0)),
                  pl.BlockSpec((1, NHID), lambda i: (0, 0)),
                  pl.BlockSpec((ROWB, NHID), lambda i: (i, 0)),
                  pl.BlockSpec((ROWB, 1), lambda i: (i, 0)),
                  pl.BlockSpec((NHID, NHID), lambda i: (0, 0)),
                  pl.BlockSpec((1, NHID), lambda i: (0, 0)),
                  pl.BlockSpec((NHID, NHID), lambda i: (0, 0)),
                  pl.BlockSpec((1, NHID), lambda i: (0, 0)),
                  pl.BlockSpec((NHID, 1), lambda i: (0, 0)),
                  pl.BlockSpec((1, 1), lambda i: (0, 0)),
                  pl.BlockSpec((NHID, 1), lambda i: (0, 0)),
                  pl.BlockSpec((1, 1), lambda i: (0, 0))],
        out_specs=[pl.BlockSpec((ROWB, 1), lambda i: (i, 0)),
                   pl.BlockSpec((ROWB, NHID), lambda i: (i, 0))],
        out_shape=[jax.ShapeDtypeStruct((N, 1), jnp.float32),
                   jax.ShapeDtypeStruct((N, NHID), jnp.float32)],
    )(sparts, zparts.reshape(NC, NP, 1), dparts.reshape(NC, NP, 1), pq,
      colsum, ro, t.reshape(N, 1), o00W,
      o00b.reshape(1, NHID), o10W, o10b.reshape(1, NHID), o01W,
      o01b.reshape(1, 1), o11W, o11b.reshape(1, 1))

    return (y2.reshape(-1), rep, treatment)


# speedup vs baseline: 1.4379x; 1.0050x over previous
import jax
import jax.numpy as jnp
from jax import lax
from jax.experimental import pallas as pl
from jax.experimental.pallas import tpu as pltpu
from jax.experimental.pallas import tpu_sc as plsc
import functools

N = 10000
NFEAT = 128
NHID = 64
E = 160000

NC, NS = 2, 16
NW = NC * NS
CH = 128
EPW = 5120
NCH = EPW // CH
E_PAD = EPW * NW
EC = E_PAD // CH
NP = 10240
RPW = NP // NS
NTRASH = NP - N
TBITS = 22
TSIZE = 1 << TBITS

ROWB = 2000

_mesh = plsc.VectorSubcoreMesh(
    core_axis_name="c", subcore_axis_name="s", num_cores=NC, num_subcores=NS)


def _tc1_body(x_ref, w_ref, o_ref):
    o_ref[...] = jnp.dot(x_ref[...], w_ref[...],
                         preferred_element_type=jnp.float32)


def _tc2_body(parts_ref, bg_ref, bgt_ref, amat_ref, ppW_ref, ppb_ref,
              pp2W_ref, pp2b_ref, ro_ref, rt_ref, pq_ref, tr_ref, cs_ref,
              ext_ref, eq_ref):
    i = pl.program_id(0)
    agg = parts_ref[0] + parts_ref[1]
    ro = jax.nn.relu(agg[:, :NHID] + bg_ref[...])
    rt = jax.nn.relu(agg[:, NHID:] + bgt_ref[...])
    ro_ref[...] = ro
    rt_ref[...] = rt
    rep = jnp.concatenate([ro, rt], axis=1)
    pq = jnp.dot(rep, amat_ref[...], preferred_element_type=jnp.float32)
    pq_ref[...] = pq
    eq = jnp.exp(pq[:, 1:2])
    eq_ref[...] = eq
    ext_ref[...] = jnp.concatenate([rt * eq, rt], axis=1)
    t1 = jnp.dot(rt, ppW_ref[...], preferred_element_type=jnp.float32)
    t1 = t1 + ppb_ref[...]
    t2 = jnp.dot(t1, pp2W_ref[...], preferred_element_type=jnp.float32)
    tr_ref[...] = jax.nn.sigmoid(t2 + pp2b_ref[...])

    @pl.when(i == 0)
    def _():
        cs_ref[...] = jnp.zeros_like(cs_ref)

    cs_ref[...] += jnp.sum(rt, axis=0, keepdims=True)


def _tc3_body(sp_ref, zs_ref, ds_ref, pq_ref, cs_ref, ro_ref, t_ref,
              o00W_ref, o00b_ref, o10W_ref, o10b_ref, o01W_ref, o01b_ref,
              o11W_ref, o11b_ref, y_ref, rep_ref):
    s = sp_ref[0] + sp_ref[1]
    ep = jnp.exp(pq_ref[...][:, :1])
    numer = ep * s[:, :NHID] - s[:, NHID:] + cs_ref[...]
    z = ep * (zs_ref[0] + zs_ref[1]) - (ds_ref[0] + ds_ref[1])
    z = z + jnp.float32(N)
    rep = numer / z + ro_ref[...]
    rep_ref[...] = rep
    y00 = jax.nn.relu(jnp.dot(rep, o00W_ref[...],
                              preferred_element_type=jnp.float32) + o00b_ref[...])
    y10 = jax.nn.relu(jnp.dot(rep, o10W_ref[...],
                              preferred_element_type=jnp.float32) + o10b_ref[...])
    y0 = jnp.dot(y00, o01W_ref[...], preferred_element_type=jnp.float32)
    y1 = jnp.dot(y10, o11W_ref[...], preferred_element_type=jnp.float32)
    y0 = y0 + o01b_ref[...]
    y1 = y1 + o11b_ref[...]
    y_ref[...] = jnp.where(t_ref[...] > 0, y1, y0)


_IOTA16 = lambda: lax.iota(jnp.int32, 16)


def _sc_dedup_body(slot_hbm, tbl_hbm, slotall, slotc, ebuf,
                   s0, s1, s2, s3, s4, s5, s6, s7):
    c = lax.axis_index("c")
    s = lax.axis_index("s")
    w = c * NS + s
    sems = (s0, s1, s2, s3, s4, s5, s6, s7)

    pltpu.sync_copy(slot_hbm.at[pl.ds(w * EPW, EPW)], slotall)
    iota16 = _IOTA16()
    base_w = w * EPW
    descs = [None] * 8
    for k in range(NCH):
        r = k % 8
        if descs[r] is not None:
            descs[r].wait()
        for g in range(CH // 16):
            dsl = pl.ds(g * 16, 16)
            slotc[r, dsl] = slotall[pl.ds(k * CH + g * 16, 16)]
            ebuf[r, dsl] = base_w + (k * CH + g * 16) + iota16
        descs[r] = pltpu.async_copy(
            ebuf.at[r], tbl_hbm.at[slotc.at[r]], sems[r])
    for r in range(8):
        if descs[r] is not None:
            descs[r].wait()


_sc_dedup = functools.partial(
    pl.kernel,
    out_type=jax.ShapeDtypeStruct((TSIZE,), jnp.int32),
    mesh=_mesh,
    scratch_types=[
        pltpu.VMEM((EPW,), jnp.int32),
        pltpu.VMEM((8, CH), jnp.int32),
        pltpu.VMEM((8, CH), jnp.int32),
        pltpu.SemaphoreType.DMA,
        pltpu.SemaphoreType.DMA,
        pltpu.SemaphoreType.DMA,
        pltpu.SemaphoreType.DMA,
        pltpu.SemaphoreType.DMA,
        pltpu.SemaphoreType.DMA,
        pltpu.SemaphoreType.DMA,
        pltpu.SemaphoreType.DMA,
    ],
)(_sc_dedup_body)


def _zero_rows3(rows, r, ncols):
    def body(i, carry):
        for g in range(ncols // 16):
            rows[r, i, pl.ds(g * 16, 16)] = jnp.zeros((16,), jnp.float32)
        return carry
    lax.fori_loop(0, CH, body, 0)


def _sc_resolve_body(tbl_hbm, slot_hbm, key_hbm, src_hbm, src2_hbm,
                     slotall, keyall, srcall, winall, kwall, sidxall,
                     semw, semk):
    c = lax.axis_index("c")
    s = lax.axis_index("s")
    w = c * NS + s
    wflat = pl.ds(w * EPW, EPW)

    pltpu.sync_copy(slot_hbm.at[wflat], slotall)
    pltpu.sync_copy(key_hbm.at[wflat], keyall)
    pltpu.sync_copy(src_hbm.at[wflat], srcall)

    def fire_win(k, carry):
        ck = pl.ds(k * CH, CH)
        pltpu.async_copy(tbl_hbm.at[slotall.at[ck]], winall.at[ck], semw)
        return carry
    lax.fori_loop(0, NCH, fire_win, 0)
    pltpu.make_async_copy(slot_hbm.at[wflat], winall, semw).wait()

    def fire_kw(k, carry):
        ck = pl.ds(k * CH, CH)
        pltpu.async_copy(key_hbm.at[winall.at[ck]], kwall.at[ck], semk)
        return carry
    lax.fori_loop(0, NCH, fire_kw, 0)
    pltpu.make_async_copy(key_hbm.at[wflat], kwall, semk).wait()

    iota16 = _IOTA16()
    base_w = w * EPW

    def resolve(k, carry):
        for g in range(CH // 16):
            dsl = pl.ds(g * 16, 16)
            fsl = pl.ds(k * CH + g * 16, 16)
            eid16 = base_w + (k * CH + g * 16) + iota16
            keep = (winall[fsl] == eid16) | (kwall[fsl] != keyall[fsl])
            trash16 = N + lax.rem(eid16, NTRASH)
            sidxall[k, dsl] = jnp.where(keep, srcall[fsl], trash16)
        return carry
    lax.fori_loop(0, NCH, resolve, 0)
    pltpu.sync_copy(sidxall, src2_hbm.at[pl.ds(w * NCH, NCH)])


_sc_resolve = functools.partial(
    pl.kernel,
    out_type=jax.ShapeDtypeStruct((EC, CH), jnp.int32),
    mesh=_mesh,
    scratch_types=[
        pltpu.VMEM((EPW,), jnp.int32),
        pltpu.VMEM((EPW,), jnp.int32),
        pltpu.VMEM((EPW,), jnp.int32),
        pltpu.VMEM((EPW,), jnp.int32),
        pltpu.VMEM((EPW,), jnp.int32),
        pltpu.VMEM((NCH, CH), jnp.int32),
        pltpu.SemaphoreType.DMA,
        pltpu.SemaphoreType.DMA,
    ],
)(_sc_resolve_body)


def _sc_seg_sum_body(h_hbm, src2_hbm, dst_hbm, out_hbm, dstall, srcall,
                     rows, agg, sr0, sr1):
    c = lax.axis_index("c")
    s = lax.axis_index("s")
    w = c * NS + s

    pltpu.sync_copy(dst_hbm.at[pl.ds(w * EPW, EPW)], dstall)
    pltpu.sync_copy(src2_hbm.at[pl.ds(w * NCH, NCH)], srcall)

    _zero_rows3(rows, 0, NFEAT)
    for kk in range(RPW // CH):
        pltpu.sync_copy(rows.at[0], agg.at[pl.ds(s * RPW + kk * CH, CH)])
    plsc.subcore_barrier()

    sems = (sr0, sr1)
    descs = [None, None]
    descs[0] = pltpu.async_copy(
        h_hbm.at[dstall.at[pl.ds(0, CH)]], rows.at[0], sems[0])
    for k in range(NCH):
        r = k & 1
        if k + 1 < NCH:
            rn = (k + 1) & 1
            descs[rn] = pltpu.async_copy(
                h_hbm.at[dstall.at[pl.ds((k + 1) * CH, CH)]],
                rows.at[rn], sems[rn])
        descs[r].wait()
        pltpu.sync_copy(rows.at[r], agg.at[srcall.at[k]], add=True)

    plsc.subcore_barrier()
    for kk in range(RPW // CH):
        pltpu.sync_copy(agg.at[pl.ds(s * RPW + kk * CH, CH)], rows.at[0])
        pltpu.sync_copy(rows.at[0], out_hbm.at[c, pl.ds(s * RPW + kk * CH, CH)])


_sc_seg_sum = functools.partial(
    pl.kernel,
    out_type=jax.ShapeDtypeStruct((NC, NP, NFEAT), jnp.float32),
    mesh=_mesh,
    scratch_types=[
        pltpu.VMEM((EPW,), jnp.int32),
        pltpu.VMEM((NCH, CH), jnp.int32),
        pltpu.VMEM((2, CH, NFEAT), jnp.float32),
        pltpu.VMEM_SHARED((NP, NFEAT), jnp.float32),
        pltpu.SemaphoreType.DMA,
        pltpu.SemaphoreType.DMA,
    ],
)(_sc_seg_sum_body)


def _sc_att_body(ext_hbm, eq_hbm, src_hbm, dst_hbm, out_hbm, zout_hbm,
                 dout_hbm, dstall, srcall, eqall, ones, rows, agg, zacc,
                 dacc, sq0, sq1, sr0, sr1):
    c = lax.axis_index("c")
    s = lax.axis_index("s")
    w = c * NS + s

    pltpu.sync_copy(dst_hbm.at[pl.ds(w * EPW, EPW)], dstall)
    pltpu.sync_copy(src_hbm.at[pl.ds(w * NCH, NCH)], srcall)

    _zero_rows3(rows, 0, NFEAT)
    for g in range(CH // 16):
        ones[pl.ds(g * 16, 16)] = jnp.ones((16,), jnp.float32)
    for kk in range(RPW // CH):
        pltpu.sync_copy(rows.at[0], agg.at[pl.ds(s * RPW + kk * CH, CH)])
        pltpu.sync_copy(rows.at[0, 0], zacc.at[pl.ds(s * RPW + kk * CH, CH)])
        pltpu.sync_copy(rows.at[0, 0], dacc.at[pl.ds(s * RPW + kk * CH, CH)])

    plsc.subcore_barrier()

    sems = (sr0, sr1)
    qsems = (sq0, sq1)
    descs = [None, None]
    qdescs = [None, None]
    descs[0] = pltpu.async_copy(
        ext_hbm.at[dstall.at[pl.ds(0, CH)]], rows.at[0], sems[0])
    qdescs[0] = pltpu.async_copy(
        eq_hbm.at[dstall.at[pl.ds(0, CH)]], eqall.at[0], qsems[0])
    for k in range(NCH):
        r = k & 1
        if k + 1 < NCH:
            rn = (k + 1) & 1
            nsl = dstall.at[pl.ds((k + 1) * CH, CH)]
            descs[rn] = pltpu.async_copy(
                ext_hbm.at[nsl], rows.at[rn], sems[rn])
            qdescs[rn] = pltpu.async_copy(
                eq_hbm.at[nsl], eqall.at[rn], qsems[rn])
        qdescs[r].wait()
        pltpu.sync_copy(eqall.at[r], zacc.at[srcall.at[k]], add=True)
        pltpu.sync_copy(ones, dacc.at[srcall.at[k]], add=True)
        descs[r].wait()
        pltpu.sync_copy(rows.at[r], agg.at[srcall.at[k]], add=True)

    plsc.subcore_barrier()
    for kk in range(RPW // CH):
        rowsl = pl.ds(s * RPW + kk * CH, CH)
        pltpu.sync_copy(agg.at[rowsl], rows.at[0])
        pltpu.sync_copy(rows.at[0], out_hbm.at[c, rowsl])
        pltpu.sync_copy(zacc.at[rowsl], eqall.at[0])
        pltpu.sync_copy(eqall.at[0], zout_hbm.at[c, rowsl])
        pltpu.sync_copy(dacc.at[rowsl], eqall.at[0])
        pltpu.sync_copy(eqall.at[0], dout_hbm.at[c, rowsl])


_sc_att = functools.partial(
    pl.kernel,
    out_type=(jax.ShapeDtypeStruct((NC, NP, NFEAT), jnp.float32),
              jax.ShapeDtypeStruct((NC, NP), jnp.float32),
              jax.ShapeDtypeStruct((NC, NP), jnp.float32)),
    mesh=_mesh,
    scratch_types=[
        pltpu.VMEM((EPW,), jnp.int32),
        pltpu.VMEM((NCH, CH), jnp.int32),
        pltpu.VMEM((2, CH), jnp.float32),
        pltpu.VMEM((CH,), jnp.float32),
        pltpu.VMEM((2, CH, NFEAT), jnp.float32),
        pltpu.VMEM_SHARED((NP, NFEAT), jnp.float32),
        pltpu.VMEM_SHARED((NP,), jnp.float32),
        pltpu.VMEM_SHARED((NP,), jnp.float32),
        pltpu.SemaphoreType.DMA,
        pltpu.SemaphoreType.DMA,
        pltpu.SemaphoreType.DMA,
        pltpu.SemaphoreType.DMA,
    ],
)(_sc_att_body)


def kernel(x, adj, t, Wg, bg, Wgt, bgt, a, ppW, ppb, pp2W, pp2b,
           o00W, o00b, o10W, o10b, o01W, o01b, o11W, o11b):
    src = adj[0]
    dst = adj[1]

    pad = jnp.arange(E_PAD - E, dtype=jnp.int32)
    key = jnp.concatenate([src * N + dst, N * N + pad])
    hh = (key.astype(jnp.uint32) * jnp.uint32(2654435761)) >> jnp.uint32(4)
    slot = (hh & jnp.uint32(TSIZE - 1)).astype(jnp.int32)
    src_p = jnp.concatenate([src, N + (pad % NTRASH)])
    dst_p = jnp.concatenate([dst, pad % N]).astype(jnp.int32)

    tbl = _sc_dedup(slot)
    src2res = _sc_resolve(tbl, slot, key, src_p)

    Wcat = jnp.concatenate([Wg, Wgt], axis=1)
    h = pl.pallas_call(
        _tc1_body,
        grid=(N // ROWB,),
        in_specs=[pl.BlockSpec((ROWB, NFEAT), lambda i: (i, 0)),
                  pl.BlockSpec((NFEAT, NFEAT), lambda i: (0, 0))],
        out_specs=pl.BlockSpec((ROWB, NFEAT), lambda i: (i, 0)),
        out_shape=jax.ShapeDtypeStruct((N, NFEAT), jnp.float32),
    )(x, Wcat)

    parts = _sc_seg_sum(h, src2res, dst_p)

    amat = jnp.concatenate([a[:NFEAT], a[NFEAT:]], axis=1)
    ro, rt, pq, treatment, colsum, ext, eq = pl.pallas_call(
        _tc2_body,
        grid=(N // ROWB,),
        in_specs=[pl.BlockSpec((NC, ROWB, NFEAT), lambda i: (0, i, 0)),
                  pl.BlockSpec((1, NHID), lambda i: (0, 0)),
                  pl.BlockSpec((1, NHID), lambda i: (0, 0)),
                  pl.BlockSpec((NFEAT, 2), lambda i: (0, 0)),
                  pl.BlockSpec((NHID, NHID), lambda i: (0, 0)),
                  pl.BlockSpec((1, NHID), lambda i: (0, 0)),
                  pl.BlockSpec((NHID, 2), lambda i: (0, 0)),
                  pl.BlockSpec((1, 2), lambda i: (0, 0))],
        out_specs=[pl.BlockSpec((ROWB, NHID), lambda i: (i, 0)),
                   pl.BlockSpec((ROWB, NHID), lambda i: (i, 0)),
                   pl.BlockSpec((ROWB, 2), lambda i: (i, 0)),
                   pl.BlockSpec((ROWB, 2), lambda i: (i, 0)),
                   pl.BlockSpec((1, NHID), lambda i: (0, 0)),
                   pl.BlockSpec((ROWB, NFEAT), lambda i: (i, 0)),
                   pl.BlockSpec((ROWB, 1), lambda i: (i, 0))],
        out_shape=[jax.ShapeDtypeStruct((N, NHID), jnp.float32),
                   jax.ShapeDtypeStruct((N, NHID), jnp.float32),
                   jax.ShapeDtypeStruct((N, 2), jnp.float32),
                   jax.ShapeDtypeStruct((N, 2), jnp.float32),
                   jax.ShapeDtypeStruct((1, NHID), jnp.float32),
                   jax.ShapeDtypeStruct((N, NFEAT), jnp.float32),
                   jax.ShapeDtypeStruct((N, 1), jnp.float32)],
    )(parts, bg.reshape(1, NHID), bgt.reshape(1, NHID), amat, ppW,
      ppb.reshape(1, NHID), pp2W, pp2b.reshape(1, 2))

    sparts, zparts, dparts = _sc_att(ext, eq.reshape(-1), src2res, dst_p)

    y2, rep = pl.pallas_call(
        _tc3_body,
        grid=(N // ROWB,),
        in_specs=[pl.BlockSpec((NC, ROWB, NFEAT), lambda i: (0, i, 0)),
                  pl.BlockSpec((NC, ROWB, 1), lambda i: (0, i, 0)),
                  pl.BlockSpec((NC, ROWB, 1), lambda i: (0, i, 0)),
                  pl.BlockSpec((ROWB, 2), lambda i: (i, 0)),
                  pl.BlockSpec((1, NHID), lambda i: (0, 0)),
                  pl.BlockSpec((ROWB, NHID), lambda i: (i, 0)),
                  pl.BlockSpec((ROWB, 1), lambda i: (i, 0)),
                  pl.BlockSpec((NHID, NHID), lambda i: (0, 0)),
                  pl.BlockSpec((1, NHID), lambda i: (0, 0)),
                  pl.BlockSpec((NHID, NHID), lambda i: (0, 0)),
                  pl.BlockSpec((1, NHID), lambda i: (0, 0)),
                  pl.BlockSpec((NHID, 1), lambda i: (0, 0)),
                  pl.BlockSpec((1, 1), lambda i: (0, 0)),
                  pl.BlockSpec((NHID, 1), lambda i: (0, 0)),
                  pl.BlockSpec((1, 1), lambda i: (0, 0))],
        out_specs=[pl.BlockSpec((ROWB, 1), lambda i: (i, 0)),
                   pl.BlockSpec((ROWB, NHID), lambda i: (i, 0))],
        out_shape=[jax.ShapeDtypeStruct((N, 1), jnp.float32),
                   jax.ShapeDtypeStruct((N, NHID), jnp.float32)],
    )(sparts, zparts.reshape(NC, NP, 1), dparts.reshape(NC, NP, 1), pq,
      colsum, ro, t.reshape(N, 1), o00W,
      o00b.reshape(1, NHID), o10W, o10b.reshape(1, NHID), o01W,
      o01b.reshape(1, 1), o11W, o11b.reshape(1, 1))

    return (y2.reshape(-1), rep, treatment)
